# baseline (speedup 1.0000x reference)
import functools

import jax
import jax.numpy as jnp
from jax import lax
from jax.experimental import pallas as pl
from jax.experimental.pallas import tpu as pltpu

_N, _D, _H, _E, _K = 2048, 1024, 2048, 8, 2
_BLK = 256


def _dot_t(a, b):
    return lax.dot_general(a, b, (((1,), (1,)), ((), ())),
                           preferred_element_type=jnp.float32)


def _moe_body(z_ref, wg_ref, bg_ref, w1_ref, b1_ref, w2_ref, b2_ref, out_ref,
              gate_ref):
    e = pl.program_id(0)
    n = pl.program_id(1)
    rows = pl.ds(n * _BLK, _BLK)
    z32 = z_ref[rows, :]
    zb = z32.astype(jnp.bfloat16)

    @pl.when(e == 0)
    def _gating():
        logits = _dot_t(zb, wg_ref[...]) + bg_ref[...]
        m = jnp.max(logits, axis=-1, keepdims=True)
        ex = jnp.exp(logits - m)
        w = ex / jnp.sum(ex, axis=-1, keepdims=True)

        eidx = lax.broadcasted_iota(jnp.int32, (_BLK, _E), 1)
        m1 = jnp.max(w, axis=-1, keepdims=True)
        i1 = jnp.min(jnp.where(w == m1, eidx, _E), axis=-1, keepdims=True)
        sel1 = eidx == i1
        wm = jnp.where(sel1, -jnp.inf, w)
        m2 = jnp.max(wm, axis=-1, keepdims=True)
        i2 = jnp.min(jnp.where(wm == m2, eidx, _E), axis=-1, keepdims=True)
        sel2 = eidx == i2
        gate_ref[rows, :] = jnp.where(sel1 | sel2, w, 0.0)

    eidx = lax.broadcasted_iota(jnp.int32, (_BLK, _E), 1)
    gcol = jnp.sum(jnp.where(eidx == e, gate_ref[rows, :], 0.0),
                   axis=-1, keepdims=True)

    h = _dot_t(zb, w1_ref[0]) + b1_ref[0]
    hb = jnp.maximum(h, 0.0).astype(jnp.bfloat16)
    y = _dot_t(hb, w2_ref[0]) + b2_ref[0]
    contrib = gcol * y

    @pl.when(e == 0)
    def _init():
        out_ref[rows, :] = contrib

    @pl.when(e != 0)
    def _acc():
        out_ref[rows, :] += contrib


@jax.jit
def kernel(z, Wg, bg, W1, b1, W2, b2):
    w1b = W1.astype(jnp.bfloat16)
    w2b = W2.astype(jnp.bfloat16)
    bg2 = bg.reshape(1, _E)
    b1r = b1.reshape(_E, 1, _H)
    b2r = b2.reshape(_E, 1, _D)

    grid = (_E, _N // _BLK)
    out = pl.pallas_call(
        _moe_body,
        grid=grid,
        in_specs=[
            pl.BlockSpec((_N, _D), lambda e, n: (0, 0)),
            pl.BlockSpec((_E, _D), lambda e, n: (0, 0)),
            pl.BlockSpec((1, _E), lambda e, n: (0, 0)),
            pl.BlockSpec((1, _H, _D), lambda e, n: (e, 0, 0)),
            pl.BlockSpec((1, 1, _H), lambda e, n: (e, 0, 0)),
            pl.BlockSpec((1, _D, _H), lambda e, n: (e, 0, 0)),
            pl.BlockSpec((1, 1, _D), lambda e, n: (e, 0, 0)),
        ],
        out_specs=pl.BlockSpec((_N, _D), lambda e, n: (0, 0)),
        out_shape=jax.ShapeDtypeStruct((_N, _D), jnp.float32),
        scratch_shapes=[pltpu.VMEM((_N, _E), jnp.float32)],
        compiler_params=pltpu.CompilerParams(
            dimension_semantics=("arbitrary", "arbitrary"),
        ),
    )(z, Wg.astype(jnp.bfloat16), bg2, w1b, b1r, w2b, b2r)
    return out


# speedup vs baseline: 1.2853x; 1.2853x over previous
import functools

import jax
import jax.numpy as jnp
from jax import lax
from jax.experimental import pallas as pl
from jax.experimental.pallas import tpu as pltpu
from jax.experimental.pallas import tpu_sc as plsc

_N, _D, _H, _E, _K = 2048, 1024, 2048, 8, 2
_BLK = 256
_NP = _N * _K
_MAXT = 23
_TOT = _MAXT * _BLK
_NBLK = _N // _BLK

_NC, _NS, _L = 2, 16, 16
_NW = _NC * _NS
_TOKS_W = _N // _NW


def _dot_t(a, b):
    return lax.dot_general(a, b, (((1,), (1,)), ((), ())),
                           preferred_element_type=jnp.float32)


def _gate_route_body(z_ref, wg_ref, bg_ref, tri_ref, dest_ref, te_ref,
                     topw_ref):
    zb = z_ref[...].astype(jnp.bfloat16)
    logits = _dot_t(zb, wg_ref[...]) + bg_ref[...]
    m = jnp.max(logits, axis=-1, keepdims=True)
    ex = jnp.exp(logits - m)
    w = ex / jnp.sum(ex, axis=-1, keepdims=True)

    eidx = lax.broadcasted_iota(jnp.int32, (_N, _E), 1)
    m1 = jnp.max(w, axis=-1, keepdims=True)
    i1 = jnp.min(jnp.where(w == m1, eidx, _E), axis=-1, keepdims=True)
    sel1 = eidx == i1
    wm = jnp.where(sel1, -jnp.inf, w)
    m2 = jnp.max(wm, axis=-1, keepdims=True)
    i2 = jnp.min(jnp.where(wm == m2, eidx, _E), axis=-1, keepdims=True)
    sel2 = eidx == i2
    topw_ref[...] = jnp.concatenate([m1, m2], axis=1)

    counts = (sel1 | sel2).astype(jnp.bfloat16)
    tri = tri_ref[...]
    s_blocks = []
    carry = jnp.zeros((1, _E), jnp.float32)
    for b in range(_NBLK):
        blk = counts[b * _BLK:(b + 1) * _BLK, :]
        p = lax.dot_general(tri, blk, (((1,), (0,)), ((), ())),
                            preferred_element_type=jnp.float32)
        s_blocks.append(p + carry)
        carry = carry + jnp.sum(blk.astype(jnp.float32), axis=0,
                                keepdims=True)
    s = jnp.concatenate(s_blocks, axis=0)
    cnt = carry
    ntiles = (cnt.astype(jnp.int32) + (_BLK - 1)) // _BLK
    tri8 = tri_ref[:8, :8]
    starts = lax.dot_general(
        ntiles.astype(jnp.bfloat16), tri8, (((1,), (1,)), ((), ())),
        preferred_element_type=jnp.float32)
    starts_i = starts.astype(jnp.int32)
    segbase = (starts_i * _BLK).astype(jnp.float32)

    rank1 = jnp.sum(jnp.where(sel1, s, 0.0), axis=-1, keepdims=True)
    base1 = jnp.sum(jnp.where(sel1, segbase, 0.0), axis=-1, keepdims=True)
    rank2 = jnp.sum(jnp.where(sel2, s, 0.0), axis=-1, keepdims=True)
    base2 = jnp.sum(jnp.where(sel2, segbase, 0.0), axis=-1, keepdims=True)
    d1 = (rank1 + base1).astype(jnp.int32)
    d2 = (rank2 + base2).astype(jnp.int32)
    zero = jnp.zeros((_N, 1), jnp.int32)
    dcols = jnp.concatenate([d1, d2] + [zero] * 6, axis=1)
    dest_ref[...] = dcols.T

    jiota = lax.broadcasted_iota(jnp.int32, (8, 32), 1)
    acc = jnp.zeros((8, 32), jnp.int32)
    for e in range(_E):
        acc = acc + (jiota >= starts_i[0, e]).astype(jnp.int32)
    te_ref[...] = acc - 1


def _gate_route(z, wgb, bg2, tri):
    return pl.pallas_call(
        _gate_route_body,
        grid=(1,),
        in_specs=[
            pl.BlockSpec((_N, _D), lambda i: (0, 0)),
            pl.BlockSpec((_E, _D), lambda i: (0, 0)),
            pl.BlockSpec((1, _E), lambda i: (0, 0)),
            pl.BlockSpec((_BLK, _BLK), lambda i: (0, 0)),
        ],
        out_specs=[
            pl.BlockSpec((8, _N), lambda i: (0, 0)),
            pl.BlockSpec((8, 32), lambda i: (0, 0)),
            pl.BlockSpec((_N, 2), lambda i: (0, 0)),
        ],
        out_shape=[
            jax.ShapeDtypeStruct((8, _N), jnp.int32),
            jax.ShapeDtypeStruct((8, 32), jnp.int32),
            jax.ShapeDtypeStruct((_N, 2), jnp.float32),
        ],
    )(z, wgb, bg2, tri)


def _sc_scatter_body(z_hbm, dest_hbm, zbuf_hbm, eidx_v, oidx_v, rows_v, sem):
    wid = lax.axis_index("s") * _NC + lax.axis_index("c")
    base_tok = wid * _TOKS_W
    pltpu.sync_copy(dest_hbm.at[0, pl.ds(base_tok, _TOKS_W)], eidx_v)
    pltpu.sync_copy(dest_hbm.at[1, pl.ds(base_tok, _TOKS_W)], oidx_v)
    pltpu.sync_copy(z_hbm.at[pl.ds(base_tok, _TOKS_W)], rows_v)
    pltpu.async_copy(rows_v, zbuf_hbm.at[eidx_v], sem).wait()
    pltpu.async_copy(rows_v, zbuf_hbm.at[oidx_v], sem).wait()


@functools.cache
def _get_sc_scatter():
    mesh = plsc.VectorSubcoreMesh(core_axis_name="c", subcore_axis_name="s")
    return functools.partial(
        pl.kernel, mesh=mesh,
        out_type=jax.ShapeDtypeStruct((_TOT, _D), jnp.float32),
        scratch_types=[
            pltpu.VMEM((_TOKS_W,), jnp.int32),
            pltpu.VMEM((_TOKS_W,), jnp.int32),
            pltpu.VMEM((_TOKS_W, _D), jnp.float32),
            pltpu.SemaphoreType.DMA,
        ],
    )(_sc_scatter_body)


def _ffn_body(te_ref, zbuf_ref, w1_ref, b1_ref, w2_ref, b2_ref, y_ref):
    zb = zbuf_ref[...].astype(jnp.bfloat16)
    h = _dot_t(zb, w1_ref[0]) + b1_ref[0]
    hb = jnp.maximum(h, 0.0).astype(jnp.bfloat16)
    y_ref[...] = _dot_t(hb, w2_ref[0]) + b2_ref[0]


def _ffn(te, zbuf, w1b, b1r, w2b, b2r):
    grid_spec = pltpu.PrefetchScalarGridSpec(
        num_scalar_prefetch=1,
        grid=(_MAXT,),
        in_specs=[
            pl.BlockSpec((_BLK, _D), lambda i, te: (i, 0)),
            pl.BlockSpec((1, _H, _D), lambda i, te: (te[i], 0, 0)),
            pl.BlockSpec((1, 1, _H), lambda i, te: (te[i], 0, 0)),
            pl.BlockSpec((1, _D, _H), lambda i, te: (te[i], 0, 0)),
            pl.BlockSpec((1, 1, _D), lambda i, te: (te[i], 0, 0)),
        ],
        out_specs=pl.BlockSpec((_BLK, _D), lambda i, te: (i, 0)),
    )
    return pl.pallas_call(
        _ffn_body,
        grid_spec=grid_spec,
        out_shape=jax.ShapeDtypeStruct((_TOT, _D), jnp.float32),
        compiler_params=pltpu.CompilerParams(
            dimension_semantics=("arbitrary",),
        ),
    )(te, zbuf, w1b, b1r, w2b, b2r)


def _sc_gather_body(ybuf_hbm, dest_hbm, g0_hbm, g1_hbm, idx_v, g_v, sem):
    wid = lax.axis_index("s") * _NC + lax.axis_index("c")
    base_tok = wid * _TOKS_W
    rows = pl.ds(base_tok, _TOKS_W)
    pltpu.sync_copy(dest_hbm.at[0, rows], idx_v)
    pltpu.async_copy(ybuf_hbm.at[idx_v], g_v, sem).wait()
    pltpu.sync_copy(g_v, g0_hbm.at[rows])
    pltpu.sync_copy(dest_hbm.at[1, rows], idx_v)
    pltpu.async_copy(ybuf_hbm.at[idx_v], g_v, sem).wait()
    pltpu.sync_copy(g_v, g1_hbm.at[rows])


@functools.cache
def _get_sc_gather():
    mesh = plsc.VectorSubcoreMesh(core_axis_name="c", subcore_axis_name="s")
    return functools.partial(
        pl.kernel, mesh=mesh,
        out_type=[
            jax.ShapeDtypeStruct((_N, _D), jnp.float32),
            jax.ShapeDtypeStruct((_N, _D), jnp.float32),
        ],
        scratch_types=[
            pltpu.VMEM((_TOKS_W,), jnp.int32),
            pltpu.VMEM((_TOKS_W, _D), jnp.float32),
            pltpu.SemaphoreType.DMA,
        ],
    )(_sc_gather_body)


def _finish_body(g0_ref, g1_ref, topw_ref, out_ref):
    w = topw_ref[...]
    out_ref[...] = (w[:, 0:1] * g0_ref[...] + w[:, 1:2] * g1_ref[...])


def _finish(g0, g1, topw2):
    return pl.pallas_call(
        _finish_body,
        grid=(_N // _BLK,),
        in_specs=[
            pl.BlockSpec((_BLK, _D), lambda i: (i, 0)),
            pl.BlockSpec((_BLK, _D), lambda i: (i, 0)),
            pl.BlockSpec((_BLK, 2), lambda i: (i, 0)),
        ],
        out_specs=pl.BlockSpec((_BLK, _D), lambda i: (i, 0)),
        out_shape=jax.ShapeDtypeStruct((_N, _D), jnp.float32),
    )(g0, g1, topw2)


@jax.jit
def kernel(z, Wg, bg, W1, b1, W2, b2):
    w1b = W1.astype(jnp.bfloat16)
    w2b = W2.astype(jnp.bfloat16)
    wgb = Wg.astype(jnp.bfloat16)
    bg2 = bg.reshape(1, _E)
    b1r = b1.reshape(_E, 1, _H)
    b2r = b2.reshape(_E, 1, _D)
    tri = jnp.tril(jnp.ones((_BLK, _BLK), jnp.bfloat16), -1)

    dest8, te8, topw2 = _gate_route(z, wgb, bg2, tri)
    te = te8[0, :_MAXT]

    zbuf = _get_sc_scatter()(z, dest8)
    ybuf = _ffn(te, zbuf, w1b, b1r, w2b, b2r)
    g0, g1 = _get_sc_gather()(ybuf, dest8)
    return _finish(g0, g1, topw2)
